# baseline (speedup 1.0000x reference)
import functools

import jax
import jax.numpy as jnp
import numpy as np
from jax import lax
from jax.experimental import pallas as pl
from jax.experimental.pallas import tpu as pltpu
from jax.experimental.pallas import tpu_sc as plsc

NNODES = 10000
NREL = 16
NEMB = 64
R_TOTAL = 2 * NREL + 1
N_EDGES = 320000
N_TRIPLES = 16384

NDIR = 2 * N_EDGES
NSEG = 2 * NREL * NNODES
NC, NS, NLANE = 2, 16, 16
NW = NC * NS
EPT = NDIR // NW
EB = 80
SPT = NSEG // NS
NPAD = 10240
RPT = NPAD // NS
TPT = N_TRIPLES // NW
TB = 128

_MESH = plsc.VectorSubcoreMesh(core_axis_name="c", subcore_axis_name="s")
_SC_PARAMS = pltpu.CompilerParams(use_tc_tiling_on_sc=False)

_BN = 10000
_GN = NNODES // _BN


def _bcast_lane(v, j):
  dnums = lax.GatherDimensionNumbers(
      offset_dims=(), collapsed_slice_dims=(0,), start_index_map=(0,))
  idx = jnp.full((NLANE, 1), j, jnp.int32)
  return lax.gather(v, idx, dnums, (1,),
                    mode=lax.GatherScatterMode.PROMISE_IN_BOUNDS)


NPAIR = NNODES // 2


def _mm1_body(emb_ref, bias_ref, w_ref, wsl_ref, y_ref, ysl_ref):
  x = jnp.maximum(emb_ref[...] + bias_ref[...], 0.0)
  y_ref[...] = jnp.dot(x, w_ref[0], preferred_element_type=jnp.float32)

  @pl.when(pl.program_id(0) == R_TOTAL - 1)
  def _():
    ysl_ref[...] = jnp.dot(x, wsl_ref[...], preferred_element_type=jnp.float32)


def _mm1(emb_p, bias2, WP, WslP):
  return pl.pallas_call(
      _mm1_body,
      grid=(R_TOTAL,),
      in_specs=[
          pl.BlockSpec((NPAIR, 2 * NEMB), lambda r: (0, 0)),
          pl.BlockSpec((1, 2 * NEMB), lambda r: (0, 0)),
          pl.BlockSpec((1, 2 * NEMB, 2 * NEMB), lambda r: (r, 0, 0)),
          pl.BlockSpec((2 * NEMB, 2 * NEMB), lambda r: (0, 0)),
      ],
      out_specs=[
          pl.BlockSpec((NPAIR, 2 * NEMB), lambda r: (r, 0)),
          pl.BlockSpec((NPAIR, 2 * NEMB), lambda r: (0, 0)),
      ],
      out_shape=[
          jax.ShapeDtypeStruct((R_TOTAL * NPAIR, 2 * NEMB), jnp.float32),
          jax.ShapeDtypeStruct((NPAIR, 2 * NEMB), jnp.float32),
      ],
  )(emb_p, bias2, WP, WslP)


def _mm2_body(part_ref, ysl1_ref, b_ref, w_ref, wsl_ref, y_ref, ysl_ref):
  x = part_ref[0] + part_ref[1] + ysl1_ref[...] + b_ref[...]
  x = jnp.maximum(x, 0.0)
  y_ref[...] = jnp.dot(x, w_ref[0], preferred_element_type=jnp.float32)

  @pl.when(pl.program_id(0) == R_TOTAL - 1)
  def _():
    ysl_ref[...] = jnp.dot(x, wsl_ref[...], preferred_element_type=jnp.float32)


def _mm2(part_p, ysl1, b2x, WP, WslP):
  return pl.pallas_call(
      _mm2_body,
      grid=(R_TOTAL,),
      in_specs=[
          pl.BlockSpec((2, NPAIR, 2 * NEMB), lambda r: (0, 0, 0)),
          pl.BlockSpec((NPAIR, 2 * NEMB), lambda r: (0, 0)),
          pl.BlockSpec((1, 2 * NEMB), lambda r: (0, 0)),
          pl.BlockSpec((1, 2 * NEMB, 2 * NEMB), lambda r: (r, 0, 0)),
          pl.BlockSpec((2 * NEMB, 2 * NEMB), lambda r: (0, 0)),
      ],
      out_specs=[
          pl.BlockSpec((NPAIR, 2 * NEMB), lambda r: (r, 0)),
          pl.BlockSpec((NPAIR, 2 * NEMB), lambda r: (0, 0)),
      ],
      out_shape=[
          jax.ShapeDtypeStruct((R_TOTAL * NPAIR, 2 * NEMB), jnp.float32),
          jax.ShapeDtypeStruct((NPAIR, 2 * NEMB), jnp.float32),
      ],
  )(part_p, ysl1, b2x.reshape(1, 2 * NEMB), WP, WslP)


def _fin_body(part_ref, ysl_ref, b_ref, x_ref):
  x_ref[...] = part_ref[0] + part_ref[1] + ysl_ref[...] + b_ref[...]


def _fin(part_p, ysl2, b2x):
  return pl.pallas_call(
      _fin_body,
      grid=(1,),
      in_specs=[
          pl.BlockSpec((2, NPAIR, 2 * NEMB), lambda n: (0, 0, 0)),
          pl.BlockSpec((NPAIR, 2 * NEMB), lambda n: (0, 0)),
          pl.BlockSpec((1, 2 * NEMB), lambda n: (0, 0)),
      ],
      out_specs=pl.BlockSpec((NPAIR, 2 * NEMB), lambda n: (0, 0)),
      out_shape=jax.ShapeDtypeStruct((NPAIR, 2 * NEMB), jnp.float32),
  )(part_p, ysl2, b2x.reshape(1, 2 * NEMB))


def _norm_body(c_ref, n_ref):
  c = c_ref[0] + c_ref[1]
  n_ref[...] = 1.0 / jnp.maximum(c, 1.0)


def _norm_tc(counts):
  out = pl.pallas_call(
      _norm_body,
      in_specs=[pl.BlockSpec((2, 2500, 128), lambda: (0, 0, 0))],
      out_specs=pl.BlockSpec((2500, 128), lambda: (0, 0)),
      out_shape=jax.ShapeDtypeStruct((2500, 128), jnp.float32),
  )(counts.reshape(2, 2500, 128))
  return out.reshape(NSEG)


_SB = 512
_SG = N_TRIPLES // _SB


def _score_body(xs_ref, xo_ref, tp_ref, rel_ref, s_ref, p_ref):
  tp = tp_ref[0, 0]
  oh = (tp[:, None] == lax.broadcasted_iota(jnp.int32, (_SB, NREL), 1))
  rl = jnp.dot(oh.astype(jnp.float32), rel_ref[...],
               preferred_element_type=jnp.float32)
  s = jnp.sum(xs_ref[0] * xo_ref[0] * rl, axis=-1)
  s_ref[...] = s[None, None, :]
  p_ref[...] = jnp.sum(rel_ref[...] ** 2).reshape(1, 1)


def _score(xs, xo, tp, relations):
  scores, penalty = pl.pallas_call(
      _score_body,
      grid=(_SG,),
      in_specs=[
          pl.BlockSpec((1, _SB, NEMB), lambda i: (i, 0, 0)),
          pl.BlockSpec((1, _SB, NEMB), lambda i: (i, 0, 0)),
          pl.BlockSpec((1, 1, _SB), lambda i: (i, 0, 0)),
          pl.BlockSpec((NREL, NEMB), lambda i: (0, 0)),
      ],
      out_specs=[
          pl.BlockSpec((1, 1, _SB), lambda i: (i, 0, 0)),
          pl.BlockSpec((1, 1), lambda i: (0, 0)),
      ],
      out_shape=[
          jax.ShapeDtypeStruct((_SG, 1, _SB), jnp.float32),
          jax.ShapeDtypeStruct((1, 1), jnp.float32),
      ],
  )(xs.reshape(_SG, _SB, NEMB), xo.reshape(_SG, _SB, NEMB),
    tp.reshape(_SG, 1, _SB), relations)
  return scores.reshape(N_TRIPLES), penalty.reshape(())


_ZB = 2000
WB = 5
CH = WB * EB
NCHUNK = EPT // CH
NBROW = NDIR // EB
RPW = EPT // EB


def _counts_body(seg_hbm, out_hbm, seg_v, ones_v, zb_v, cnt_sp, sem, sem_s):
  c = lax.axis_index("c")
  s = lax.axis_index("s")
  w = s * NC + c

  @pl.loop(0, EB // NLANE)
  def _(i):
    ones_v[pl.ds(i * NLANE, NLANE)] = jnp.ones((NLANE,), jnp.float32)

  @pl.loop(0, _ZB // NLANE)
  def _(i):
    zb_v[pl.ds(i * NLANE, NLANE)] = jnp.zeros((NLANE,), jnp.float32)

  @pl.loop(0, SPT // _ZB)
  def _(i):
    pltpu.sync_copy(zb_v, cnt_sp.at[pl.ds(s * SPT + i * _ZB, _ZB)])

  plsc.subcore_barrier()

  @pl.loop(0, NCHUNK)
  def _(i):
    r0 = w * RPW + i * WB
    pltpu.sync_copy(seg_hbm.at[pl.ds(r0, WB)], seg_v)
    ds_ = [pltpu.async_copy(ones_v, cnt_sp.at[seg_v.at[b]], sem_s, add=True)
           for b in range(WB)]
    for d in ds_:
      d.wait()

  plsc.subcore_barrier()
  pltpu.sync_copy(cnt_sp.at[pl.ds(s * SPT, SPT)], out_hbm.at[c, s])


def _sc_counts(seg2):
  k = pl.kernel(
      _counts_body,
      out_type=jax.ShapeDtypeStruct((NC, NS, SPT), jnp.float32),
      mesh=_MESH,
      compiler_params=_SC_PARAMS,
      scratch_types=[
          pltpu.VMEM((WB, EB), jnp.int32),
          pltpu.VMEM((EB,), jnp.float32),
          pltpu.VMEM((_ZB,), jnp.float32),
          pltpu.VMEM_SHARED((NSEG,), jnp.float32),
          pltpu.SemaphoreType.DMA,
          pltpu.SemaphoreType.DMA,
      ],
  )
  return k(seg2)


def _enorm_body(seg_hbm, norms_hbm, out_hbm, seg_v, nrm_v, sem):
  c = lax.axis_index("c")
  s = lax.axis_index("s")
  w = s * NC + c

  @pl.loop(0, NCHUNK)
  def _(i):
    r0 = w * RPW + i * WB
    pltpu.sync_copy(seg_hbm.at[pl.ds(r0, WB)], seg_v)
    ds_ = [pltpu.async_copy(norms_hbm.at[seg_v.at[b]], nrm_v.at[b], sem)
           for b in range(WB)]
    for d in ds_:
      d.wait()
    pltpu.sync_copy(nrm_v, out_hbm.at[pl.ds(r0, WB)])


def _sc_enorm(seg2, norms):
  k = pl.kernel(
      _enorm_body,
      out_type=jax.ShapeDtypeStruct((NBROW, EB), jnp.float32),
      mesh=_MESH,
      compiler_params=_SC_PARAMS,
      scratch_types=[
          pltpu.VMEM((WB, EB), jnp.int32),
          pltpu.VMEM((WB, EB), jnp.float32),
          pltpu.SemaphoreType.DMA,
      ],
  )
  return k(seg2, norms)


def _agg_body(y_hbm, gidx_hbm, dst_hbm, nrm_hbm, out_hbm,
              gidx_v, dst_v, nrm_v, rows_v, zb_v, acc_sp,
              sem_l0, sem_l1, sem_g0, sem_g1, sem_g2, sem_g3, sem_g4,
              sem_s0, sem_s1):
  c = lax.axis_index("c")
  s = lax.axis_index("s")
  w = s * NC + c
  row_base = w * RPW
  sem_l = (sem_l0, sem_l1)
  sem_g = (sem_g0, sem_g1, sem_g2, sem_g3, sem_g4)
  sem_s = (sem_s0, sem_s1)

  @pl.loop(0, 32)
  def _(i):
    for kq in range(NEMB // NLANE):
      zb_v[i, pl.ds(kq * NLANE, NLANE)] = jnp.zeros((NLANE,), jnp.float32)

  @pl.loop(0, RPT // 32)
  def _(i):
    pltpu.sync_copy(zb_v, acc_sp.at[pl.ds(s * RPT + i * 32, 32)])

  plsc.subcore_barrier()

  def fire_linear(ci, p):
    r0 = row_base + ci * WB
    pltpu.async_copy(gidx_hbm.at[pl.ds(r0, WB)], gidx_v.at[p], sem_l[p])
    pltpu.async_copy(dst_hbm.at[pl.ds(r0, WB)], dst_v.at[p], sem_l[p])
    pltpu.async_copy(nrm_hbm.at[pl.ds(r0, WB)], nrm_v.at[p], sem_l[p])

  def drain_linear(ci, p):
    r0 = row_base + ci * WB
    pltpu.make_async_copy(gidx_hbm.at[pl.ds(r0, WB)], gidx_v.at[p],
                          sem_l[p]).wait()
    pltpu.make_async_copy(dst_hbm.at[pl.ds(r0, WB)], dst_v.at[p],
                          sem_l[p]).wait()
    pltpu.make_async_copy(nrm_hbm.at[pl.ds(r0, WB)], nrm_v.at[p],
                          sem_l[p]).wait()

  def scale_batch(p, b):
    @pl.loop(0, EB // NLANE)
    def _(gg):
      nv = nrm_v[p, b, pl.ds(gg * NLANE, NLANE)]
      for j in range(NLANE):
        nb = _bcast_lane(nv, j)
        e = gg * NLANE + j
        for kq in range(NEMB // NLANE):
          sl = pl.ds(kq * NLANE, NLANE)
          rows_v[p, b, e, sl] = rows_v[p, b, e, sl] * nb

  def drain_scatters(p):
    for b in range(WB):
      pltpu.make_async_copy(rows_v.at[p, b], acc_sp.at[dst_v.at[p, b]],
                            sem_s[p]).wait()

  def body(ci, p):
    drain_linear(ci, p)
    ds_ = [pltpu.async_copy(y_hbm.at[gidx_v.at[p, b]], rows_v.at[p, b],
                            sem_g[b]) for b in range(WB)]
    for b in range(WB):
      ds_[b].wait()
      scale_batch(p, b)
      pltpu.async_copy(rows_v.at[p, b], acc_sp.at[dst_v.at[p, b]], sem_s[p],
                       add=True)

  fire_linear(0, 0)
  fire_linear(1, 1)
  body(0, 0)

  @pl.loop(0, (NCHUNK - 2) // 2)
  def _(i):
    ci = 2 * i + 1
    for (cc, p) in ((ci, 1), (ci + 1, 0)):
      body(cc, p)
      drain_scatters(1 - p)
      fire_linear(cc + 1, 1 - p)

  body(NCHUNK - 1, 1)
  drain_scatters(0)
  drain_scatters(1)

  plsc.subcore_barrier()
  pltpu.sync_copy(acc_sp.at[pl.ds(s * RPT, RPT)], out_hbm.at[c, s])


def _sc_agg(y_flat, gidx2, dst2, nrm2):
  k = pl.kernel(
      _agg_body,
      out_type=jax.ShapeDtypeStruct((NC, NS, RPT, NEMB), jnp.float32),
      mesh=_MESH,
      compiler_params=_SC_PARAMS,
      scratch_types=[
          pltpu.VMEM((2, WB, EB), jnp.int32),
          pltpu.VMEM((2, WB, EB), jnp.int32),
          pltpu.VMEM((2, WB, EB), jnp.float32),
          pltpu.VMEM((2, WB, EB, NEMB), jnp.float32),
          pltpu.VMEM((32, NEMB), jnp.float32),
          pltpu.VMEM_SHARED((NPAD, NEMB), jnp.float32),
          pltpu.SemaphoreType.DMA,
          pltpu.SemaphoreType.DMA,
          pltpu.SemaphoreType.DMA,
          pltpu.SemaphoreType.DMA,
          pltpu.SemaphoreType.DMA,
          pltpu.SemaphoreType.DMA,
          pltpu.SemaphoreType.DMA,
          pltpu.SemaphoreType.DMA,
          pltpu.SemaphoreType.DMA,
      ],
  )
  return k(y_flat, gidx2, dst2, nrm2)


def _tri_body(x2_hbm, ts_hbm, to_hbm, xs_hbm, xo_hbm, idx_v, rows_v, sem):
  c = lax.axis_index("c")
  s = lax.axis_index("s")
  w = s * NC + c

  @pl.loop(0, TPT // TB)
  def _(i):
    base = w * TPT + i * TB
    pltpu.sync_copy(ts_hbm.at[pl.ds(base, TB)], idx_v)
    pltpu.async_copy(x2_hbm.at[idx_v], rows_v, sem).wait()
    pltpu.sync_copy(rows_v, xs_hbm.at[pl.ds(base, TB)])
    pltpu.sync_copy(to_hbm.at[pl.ds(base, TB)], idx_v)
    pltpu.async_copy(x2_hbm.at[idx_v], rows_v, sem).wait()
    pltpu.sync_copy(rows_v, xo_hbm.at[pl.ds(base, TB)])


def _sc_tri(x2, ts, to):
  k = pl.kernel(
      _tri_body,
      out_type=(jax.ShapeDtypeStruct((N_TRIPLES, NEMB), jnp.float32),
                jax.ShapeDtypeStruct((N_TRIPLES, NEMB), jnp.float32)),
      mesh=_MESH,
      compiler_params=_SC_PARAMS,
      scratch_types=[
          pltpu.VMEM((TB,), jnp.int32),
          pltpu.VMEM((TB, NEMB), jnp.float32),
          pltpu.SemaphoreType.DMA,
      ],
  )
  return k(x2, ts, to)


def kernel(node_embeddings, node_embeddings_bias, W1, b1, W2, b2, relations,
           graph, triples):
  s = graph[:, 0]
  r = graph[:, 1] % NREL
  o = graph[:, 2]
  rel = jnp.concatenate([r, r + NREL])
  src = jnp.concatenate([s, o])
  dst = jnp.concatenate([o, s])
  seg2 = (rel * NNODES + dst).reshape(NBROW, EB)
  gidx2 = (rel * NNODES + src).reshape(NBROW, EB)
  dst2 = dst.reshape(NBROW, EB)

  def blockdiag(W):
    z = jnp.zeros(W.shape[:-2] + (2 * NEMB, 2 * NEMB), jnp.float32)
    return z.at[..., :NEMB, :NEMB].set(W).at[..., NEMB:, NEMB:].set(W)

  W1P = blockdiag(W1)
  W2P = blockdiag(W2)
  emb_p = node_embeddings.reshape(NPAIR, 2 * NEMB)
  bias2 = jnp.concatenate([node_embeddings_bias, node_embeddings_bias], -1)
  b1x = jnp.concatenate([b1, b1])
  b2x = jnp.concatenate([b2, b2])

  y1, ysl1 = _mm1(emb_p, bias2, W1P, W1P[2 * NREL])
  counts = _sc_counts(seg2)
  norms = _norm_tc(counts)
  nrm2 = _sc_enorm(seg2, norms)
  part1 = _sc_agg(y1.reshape(R_TOTAL * NNODES, NEMB), gidx2, dst2, nrm2)
  part1_p = part1.reshape(NC, NPAD // 2, 2 * NEMB)
  y2, ysl2 = _mm2(part1_p, ysl1, b1x, W2P, W2P[2 * NREL])
  part2 = _sc_agg(y2.reshape(R_TOTAL * NNODES, NEMB), gidx2, dst2, nrm2)
  part2_p = part2.reshape(NC, NPAD // 2, 2 * NEMB)
  x2 = _fin(part2_p, ysl2, b2x).reshape(NNODES, NEMB)

  ts = triples[:, 0]
  tp = triples[:, 1] % NREL
  to = triples[:, 2]
  xs, xo = _sc_tri(x2, ts, to)
  scores, penalty = _score(xs, xo, tp, relations)
  return (scores, penalty)


# speedup vs baseline: 1.2750x; 1.2750x over previous
import functools

import jax
import jax.numpy as jnp
import numpy as np
from jax import lax
from jax.experimental import pallas as pl
from jax.experimental.pallas import tpu as pltpu
from jax.experimental.pallas import tpu_sc as plsc

NNODES = 10000
NREL = 16
NEMB = 64
R_TOTAL = 2 * NREL + 1
N_EDGES = 320000
N_TRIPLES = 16384

NDIR = 2 * N_EDGES
NSEG = 2 * NREL * NNODES
NC, NS, NLANE = 2, 16, 16
NW = NC * NS
EPT = NDIR // NW
EB = 80
SPT = NSEG // NS
NPAD = 10240
RPT = NPAD // NS
TPT = N_TRIPLES // NW
TB = 128

_MESH = plsc.VectorSubcoreMesh(core_axis_name="c", subcore_axis_name="s")
_SC_PARAMS = pltpu.CompilerParams(use_tc_tiling_on_sc=False)

_BN = 10000
_GN = NNODES // _BN


def _bcast_lane(v, j):
  dnums = lax.GatherDimensionNumbers(
      offset_dims=(), collapsed_slice_dims=(0,), start_index_map=(0,))
  idx = jnp.full((NLANE, 1), j, jnp.int32)
  return lax.gather(v, idx, dnums, (1,),
                    mode=lax.GatherScatterMode.PROMISE_IN_BOUNDS)


NPAIR = NNODES // 2


def _mm1_body(emb_ref, bias_ref, w_ref, wsl_ref, y_ref, ysl_ref):
  x = jnp.maximum(emb_ref[...] + bias_ref[...], 0.0)
  y_ref[...] = jnp.dot(x, w_ref[0], preferred_element_type=jnp.float32)

  @pl.when(pl.program_id(0) == R_TOTAL - 1)
  def _():
    ysl_ref[...] = jnp.dot(x, wsl_ref[...], preferred_element_type=jnp.float32)


def _mm1(emb_p, bias2, WP, WslP):
  return pl.pallas_call(
      _mm1_body,
      grid=(R_TOTAL,),
      in_specs=[
          pl.BlockSpec((NPAIR, 2 * NEMB), lambda r: (0, 0)),
          pl.BlockSpec((1, 2 * NEMB), lambda r: (0, 0)),
          pl.BlockSpec((1, 2 * NEMB, 2 * NEMB), lambda r: (r, 0, 0)),
          pl.BlockSpec((2 * NEMB, 2 * NEMB), lambda r: (0, 0)),
      ],
      out_specs=[
          pl.BlockSpec((NPAIR, 2 * NEMB), lambda r: (r, 0)),
          pl.BlockSpec((NPAIR, 2 * NEMB), lambda r: (0, 0)),
      ],
      out_shape=[
          jax.ShapeDtypeStruct((R_TOTAL * NPAIR, 2 * NEMB), jnp.float32),
          jax.ShapeDtypeStruct((NPAIR, 2 * NEMB), jnp.float32),
      ],
  )(emb_p, bias2, WP, WslP)


def _mm2_body(part_ref, ysl1_ref, b_ref, w_ref, wsl_ref, y_ref, ysl_ref):
  x = part_ref[0] + part_ref[1] + ysl1_ref[...] + b_ref[...]
  x = jnp.maximum(x, 0.0)
  y_ref[...] = jnp.dot(x, w_ref[0], preferred_element_type=jnp.float32)

  @pl.when(pl.program_id(0) == R_TOTAL - 1)
  def _():
    ysl_ref[...] = jnp.dot(x, wsl_ref[...], preferred_element_type=jnp.float32)


def _mm2(part_p, ysl1, b2x, WP, WslP):
  return pl.pallas_call(
      _mm2_body,
      grid=(R_TOTAL,),
      in_specs=[
          pl.BlockSpec((2, NPAIR, 2 * NEMB), lambda r: (0, 0, 0)),
          pl.BlockSpec((NPAIR, 2 * NEMB), lambda r: (0, 0)),
          pl.BlockSpec((1, 2 * NEMB), lambda r: (0, 0)),
          pl.BlockSpec((1, 2 * NEMB, 2 * NEMB), lambda r: (r, 0, 0)),
          pl.BlockSpec((2 * NEMB, 2 * NEMB), lambda r: (0, 0)),
      ],
      out_specs=[
          pl.BlockSpec((NPAIR, 2 * NEMB), lambda r: (r, 0)),
          pl.BlockSpec((NPAIR, 2 * NEMB), lambda r: (0, 0)),
      ],
      out_shape=[
          jax.ShapeDtypeStruct((R_TOTAL * NPAIR, 2 * NEMB), jnp.float32),
          jax.ShapeDtypeStruct((NPAIR, 2 * NEMB), jnp.float32),
      ],
  )(part_p, ysl1, b2x.reshape(1, 2 * NEMB), WP, WslP)


def _fin_body(part_ref, ysl_ref, b_ref, x_ref):
  x_ref[...] = part_ref[0] + part_ref[1] + ysl_ref[...] + b_ref[...]


def _fin(part_p, ysl2, b2x):
  return pl.pallas_call(
      _fin_body,
      grid=(1,),
      in_specs=[
          pl.BlockSpec((2, NPAIR, 2 * NEMB), lambda n: (0, 0, 0)),
          pl.BlockSpec((NPAIR, 2 * NEMB), lambda n: (0, 0)),
          pl.BlockSpec((1, 2 * NEMB), lambda n: (0, 0)),
      ],
      out_specs=pl.BlockSpec((NPAIR, 2 * NEMB), lambda n: (0, 0)),
      out_shape=jax.ShapeDtypeStruct((NPAIR, 2 * NEMB), jnp.float32),
  )(part_p, ysl2, b2x.reshape(1, 2 * NEMB))


def _norm_body(c_ref, n_ref):
  c = c_ref[0] + c_ref[1]
  n_ref[...] = 1.0 / jnp.maximum(c, 1.0)


def _norm_tc(counts):
  out = pl.pallas_call(
      _norm_body,
      in_specs=[pl.BlockSpec((2, 2500, 128), lambda: (0, 0, 0))],
      out_specs=pl.BlockSpec((2500, 128), lambda: (0, 0)),
      out_shape=jax.ShapeDtypeStruct((2500, 128), jnp.float32),
  )(counts.reshape(2, 2500, 128))
  return out.reshape(NSEG)


_SB = 512
_SG = N_TRIPLES // _SB


def _score_body(xs_ref, xo_ref, tp_ref, rel_ref, s_ref, p_ref):
  tp = tp_ref[0, 0]
  oh = (tp[:, None] == lax.broadcasted_iota(jnp.int32, (_SB, NREL), 1))
  rl = jnp.dot(oh.astype(jnp.float32), rel_ref[...],
               preferred_element_type=jnp.float32)
  s = jnp.sum(xs_ref[0] * xo_ref[0] * rl, axis=-1)
  s_ref[...] = s[None, None, :]
  p_ref[...] = jnp.sum(rel_ref[...] ** 2).reshape(1, 1)


def _score(xs, xo, tp, relations):
  scores, penalty = pl.pallas_call(
      _score_body,
      grid=(_SG,),
      in_specs=[
          pl.BlockSpec((1, _SB, NEMB), lambda i: (i, 0, 0)),
          pl.BlockSpec((1, _SB, NEMB), lambda i: (i, 0, 0)),
          pl.BlockSpec((1, 1, _SB), lambda i: (i, 0, 0)),
          pl.BlockSpec((NREL, NEMB), lambda i: (0, 0)),
      ],
      out_specs=[
          pl.BlockSpec((1, 1, _SB), lambda i: (i, 0, 0)),
          pl.BlockSpec((1, 1), lambda i: (0, 0)),
      ],
      out_shape=[
          jax.ShapeDtypeStruct((_SG, 1, _SB), jnp.float32),
          jax.ShapeDtypeStruct((1, 1), jnp.float32),
      ],
  )(xs.reshape(_SG, _SB, NEMB), xo.reshape(_SG, _SB, NEMB),
    tp.reshape(_SG, 1, _SB), relations)
  return scores.reshape(N_TRIPLES), penalty.reshape(())


_ZB = 2000
WB = 5
CH = WB * EB
NCHUNK = EPT // CH
NBROW = NDIR // EB
RPW = EPT // EB


def _counts_body(seg_hbm, out_hbm, seg_v, ones_v, zb_v, cnt_sp, sem, sem_s):
  c = lax.axis_index("c")
  s = lax.axis_index("s")
  w = s * NC + c

  @pl.loop(0, EB // NLANE)
  def _(i):
    ones_v[pl.ds(i * NLANE, NLANE)] = jnp.ones((NLANE,), jnp.float32)

  @pl.loop(0, _ZB // NLANE)
  def _(i):
    zb_v[pl.ds(i * NLANE, NLANE)] = jnp.zeros((NLANE,), jnp.float32)

  @pl.loop(0, SPT // _ZB)
  def _(i):
    pltpu.sync_copy(zb_v, cnt_sp.at[pl.ds(s * SPT + i * _ZB, _ZB)])

  plsc.subcore_barrier()

  @pl.loop(0, NCHUNK)
  def _(i):
    r0 = w * RPW + i * WB
    pltpu.sync_copy(seg_hbm.at[pl.ds(r0, WB)], seg_v)
    ds_ = [pltpu.async_copy(ones_v, cnt_sp.at[seg_v.at[b]], sem_s, add=True)
           for b in range(WB)]
    for d in ds_:
      d.wait()

  plsc.subcore_barrier()
  pltpu.sync_copy(cnt_sp.at[pl.ds(s * SPT, SPT)], out_hbm.at[c, s])


def _sc_counts(seg2):
  k = pl.kernel(
      _counts_body,
      out_type=jax.ShapeDtypeStruct((NC, NS, SPT), jnp.float32),
      mesh=_MESH,
      compiler_params=_SC_PARAMS,
      scratch_types=[
          pltpu.VMEM((WB, EB), jnp.int32),
          pltpu.VMEM((EB,), jnp.float32),
          pltpu.VMEM((_ZB,), jnp.float32),
          pltpu.VMEM_SHARED((NSEG,), jnp.float32),
          pltpu.SemaphoreType.DMA,
          pltpu.SemaphoreType.DMA,
      ],
  )
  return k(seg2)


def _enorm_body(seg_hbm, norms_hbm, out_hbm, seg_v, nrm_v, sem):
  c = lax.axis_index("c")
  s = lax.axis_index("s")
  w = s * NC + c

  @pl.loop(0, NCHUNK)
  def _(i):
    r0 = w * RPW + i * WB
    pltpu.sync_copy(seg_hbm.at[pl.ds(r0, WB)], seg_v)
    ds_ = [pltpu.async_copy(norms_hbm.at[seg_v.at[b]], nrm_v.at[b], sem)
           for b in range(WB)]
    for d in ds_:
      d.wait()
    pltpu.sync_copy(nrm_v, out_hbm.at[pl.ds(r0, WB)])


def _sc_enorm(seg2, norms):
  k = pl.kernel(
      _enorm_body,
      out_type=jax.ShapeDtypeStruct((NBROW, EB), jnp.float32),
      mesh=_MESH,
      compiler_params=_SC_PARAMS,
      scratch_types=[
          pltpu.VMEM((WB, EB), jnp.int32),
          pltpu.VMEM((WB, EB), jnp.float32),
          pltpu.SemaphoreType.DMA,
      ],
  )
  return k(seg2, norms)


def _agg_body(y_hbm, gidx_hbm, dst_hbm, nrm_hbm, out_hbm,
              gidx_v, dst_v, nrm_v, rows_v, zb_v, acc_sp,
              sem_l0, sem_l1, sem_g, sem_s0, sem_s1):
  c = lax.axis_index("c")
  s = lax.axis_index("s")
  w = s * NC + c
  row_base = w * RPW
  sem_l = (sem_l0, sem_l1)
  sem_s = (sem_s0, sem_s1)

  @pl.loop(0, 32)
  def _(i):
    for kq in range(NEMB // NLANE):
      zb_v[i, pl.ds(kq * NLANE, NLANE)] = jnp.zeros((NLANE,), jnp.float32)

  @pl.loop(0, RPT // 32)
  def _(i):
    pltpu.sync_copy(zb_v, acc_sp.at[pl.ds(s * RPT + i * 32, 32)])

  plsc.subcore_barrier()

  def fire_linear(ci, p):
    r0 = row_base + ci * WB
    pltpu.async_copy(gidx_hbm.at[pl.ds(r0, WB)], gidx_v.at[p], sem_l[p])
    pltpu.async_copy(dst_hbm.at[pl.ds(r0, WB)], dst_v.at[p], sem_l[p])
    pltpu.async_copy(nrm_hbm.at[pl.ds(r0, WB)], nrm_v.at[p], sem_l[p])

  def drain_linear(ci, p):
    r0 = row_base + ci * WB
    pltpu.make_async_copy(gidx_hbm.at[pl.ds(r0, WB)], gidx_v.at[p],
                          sem_l[p]).wait()
    pltpu.make_async_copy(dst_hbm.at[pl.ds(r0, WB)], dst_v.at[p],
                          sem_l[p]).wait()
    pltpu.make_async_copy(nrm_hbm.at[pl.ds(r0, WB)], nrm_v.at[p],
                          sem_l[p]).wait()

  def run_gathers(p):
    ds_ = [pltpu.async_copy(y_hbm.at[gidx_v.at[p, b]], rows_v.at[p, b], sem_g)
           for b in range(WB)]
    for d in ds_:
      d.wait()

  def scale(p):
    @pl.loop(0, WB * (EB // NLANE))
    def _(t):
      bb = t // (EB // NLANE)
      gg = t % (EB // NLANE)
      nv = nrm_v[p, bb, pl.ds(gg * NLANE, NLANE)]
      for j in range(NLANE):
        nb = _bcast_lane(nv, j)
        e = gg * NLANE + j
        for kq in range(NEMB // NLANE):
          sl = pl.ds(kq * NLANE, NLANE)
          rows_v[p, bb, e, sl] = rows_v[p, bb, e, sl] * nb

  def fire_scatters(p):
    for b in range(WB):
      pltpu.async_copy(rows_v.at[p, b], acc_sp.at[dst_v.at[p, b]], sem_s[p],
                       add=True)

  def drain_scatters(p):
    for b in range(WB):
      pltpu.make_async_copy(rows_v.at[p, b], acc_sp.at[dst_v.at[p, b]],
                            sem_s[p]).wait()

  def body(ci, p):
    drain_linear(ci, p)
    run_gathers(p)
    scale(p)
    fire_scatters(p)

  fire_linear(0, 0)
  fire_linear(1, 1)
  body(0, 0)

  @pl.loop(0, (NCHUNK - 2) // 2)
  def _(i):
    ci = 2 * i + 1
    for (cc, p) in ((ci, 1), (ci + 1, 0)):
      body(cc, p)
      drain_scatters(1 - p)
      fire_linear(cc + 1, 1 - p)

  body(NCHUNK - 1, 1)
  drain_scatters(0)
  drain_scatters(1)

  plsc.subcore_barrier()
  pltpu.sync_copy(acc_sp.at[pl.ds(s * RPT, RPT)], out_hbm.at[c, s])


def _sc_agg(y_flat, gidx2, dst2, nrm2):
  k = pl.kernel(
      _agg_body,
      out_type=jax.ShapeDtypeStruct((NC, NS, RPT, NEMB), jnp.float32),
      mesh=_MESH,
      compiler_params=_SC_PARAMS,
      scratch_types=[
          pltpu.VMEM((2, WB, EB), jnp.int32),
          pltpu.VMEM((2, WB, EB), jnp.int32),
          pltpu.VMEM((2, WB, EB), jnp.float32),
          pltpu.VMEM((2, WB, EB, NEMB), jnp.float32),
          pltpu.VMEM((32, NEMB), jnp.float32),
          pltpu.VMEM_SHARED((NPAD, NEMB), jnp.float32),
          pltpu.SemaphoreType.DMA,
          pltpu.SemaphoreType.DMA,
          pltpu.SemaphoreType.DMA,
          pltpu.SemaphoreType.DMA,
          pltpu.SemaphoreType.DMA,
      ],
  )
  return k(y_flat, gidx2, dst2, nrm2)


def _tri_body(x2_hbm, ts_hbm, to_hbm, xs_hbm, xo_hbm, idx_v, rows_v, sem):
  c = lax.axis_index("c")
  s = lax.axis_index("s")
  w = s * NC + c

  @pl.loop(0, TPT // TB)
  def _(i):
    base = w * TPT + i * TB
    pltpu.sync_copy(ts_hbm.at[pl.ds(base, TB)], idx_v)
    pltpu.async_copy(x2_hbm.at[idx_v], rows_v, sem).wait()
    pltpu.sync_copy(rows_v, xs_hbm.at[pl.ds(base, TB)])
    pltpu.sync_copy(to_hbm.at[pl.ds(base, TB)], idx_v)
    pltpu.async_copy(x2_hbm.at[idx_v], rows_v, sem).wait()
    pltpu.sync_copy(rows_v, xo_hbm.at[pl.ds(base, TB)])


def _sc_tri(x2, ts, to):
  k = pl.kernel(
      _tri_body,
      out_type=(jax.ShapeDtypeStruct((N_TRIPLES, NEMB), jnp.float32),
                jax.ShapeDtypeStruct((N_TRIPLES, NEMB), jnp.float32)),
      mesh=_MESH,
      compiler_params=_SC_PARAMS,
      scratch_types=[
          pltpu.VMEM((TB,), jnp.int32),
          pltpu.VMEM((TB, NEMB), jnp.float32),
          pltpu.SemaphoreType.DMA,
      ],
  )
  return k(x2, ts, to)


def kernel(node_embeddings, node_embeddings_bias, W1, b1, W2, b2, relations,
           graph, triples):
  s = graph[:, 0]
  r = graph[:, 1] % NREL
  o = graph[:, 2]
  rel = jnp.concatenate([r, r + NREL])
  src = jnp.concatenate([s, o])
  dst = jnp.concatenate([o, s])
  seg2 = (rel * NNODES + dst).reshape(NBROW, EB)
  gidx2 = (rel * NNODES + src).reshape(NBROW, EB)
  dst2 = dst.reshape(NBROW, EB)

  def blockdiag(W):
    z = jnp.zeros(W.shape[:-2] + (2 * NEMB, 2 * NEMB), jnp.float32)
    return z.at[..., :NEMB, :NEMB].set(W).at[..., NEMB:, NEMB:].set(W)

  W1P = blockdiag(W1)
  W2P = blockdiag(W2)
  emb_p = node_embeddings.reshape(NPAIR, 2 * NEMB)
  bias2 = jnp.concatenate([node_embeddings_bias, node_embeddings_bias], -1)
  b1x = jnp.concatenate([b1, b1])
  b2x = jnp.concatenate([b2, b2])

  y1, ysl1 = _mm1(emb_p, bias2, W1P, W1P[2 * NREL])
  counts = _sc_counts(seg2)
  norms = _norm_tc(counts)
  nrm2 = _sc_enorm(seg2, norms)
  part1 = _sc_agg(y1.reshape(R_TOTAL * NNODES, NEMB), gidx2, dst2, nrm2)
  part1_p = part1.reshape(NC, NPAD // 2, 2 * NEMB)
  y2, ysl2 = _mm2(part1_p, ysl1, b1x, W2P, W2P[2 * NREL])
  part2 = _sc_agg(y2.reshape(R_TOTAL * NNODES, NEMB), gidx2, dst2, nrm2)
  part2_p = part2.reshape(NC, NPAD // 2, 2 * NEMB)
  x2 = _fin(part2_p, ysl2, b2x).reshape(NNODES, NEMB)

  ts = triples[:, 0]
  tp = triples[:, 1] % NREL
  to = triples[:, 2]
  xs, xo = _sc_tri(x2, ts, to)
  scores, penalty = _score(xs, xo, tp, relations)
  return (scores, penalty)
